# baseline (speedup 1.0000x reference)
import functools
import jax
import jax.numpy as jnp
from jax import lax
from jax.experimental import pallas as pl
from jax.experimental.pallas import tpu as pltpu
from jax.experimental.pallas import tpu_sc as plsc

_NC = 2
_NS = 16
_NW = _NC * _NS
_CH = 80
_G = 64

_f32 = jnp.float32


def _sc_mesh():
  return plsc.VectorSubcoreMesh(
      core_axis_name="c", subcore_axis_name="s",
      num_cores=_NC, num_subcores=_NS)


def _zero_1d(ref, n):
  def body(i, carry):
    ref[pl.ds(i * 16, 16)] = jnp.zeros((16,), ref.dtype)
    return carry
  lax.fori_loop(0, n // 16, body, 0)


def _fill_1d(ref, n, val):
  def body(i, carry):
    ref[pl.ds(i * 16, 16)] = jnp.full((16,), val, ref.dtype)
    return carry
  lax.fori_loop(0, n // 16, body, 0)
  if n % 16:
    ref[pl.ds(n - 16, 16)] = jnp.full((16,), val, ref.dtype)


def _zero_2d(ref, rows):
  def body(i, carry):
    for k in range(8):
      ref[i, pl.ds(k * 16, 16)] = jnp.zeros((16,), ref.dtype)
    return carry
  lax.fori_loop(0, rows, body, 0)


def _make_deg_kernel(n, kc):
  @functools.partial(
      pl.kernel,
      out_type=(jax.ShapeDtypeStruct((n,), _f32),
                jax.ShapeDtypeStruct((n,), _f32)),
      mesh=_sc_mesh(),
      scratch_types=[
          pltpu.VMEM_SHARED((n,), _f32),
          pltpu.VMEM((kc, _CH), jnp.int32),
          pltpu.VMEM((640,), _f32),
          pltpu.VMEM((_CH,), _f32),
          [pltpu.SemaphoreType.DMA] * 4,
      ],
  )
  def deg_kernel(dst3, out0, out1, deg_sp, dst_v, zbuf, ones, u):
    c = lax.axis_index("c")
    s = lax.axis_index("s")
    wid = s * _NC + c
    _zero_1d(zbuf, 640)
    _fill_1d(ones, _CH, 1.0)
    start = jnp.minimum(s * 640, n - 640)
    pltpu.sync_copy(zbuf, deg_sp.at[pl.ds(start, 640)])
    pltpu.sync_copy(dst3.at[wid], dst_v)
    plsc.subcore_barrier()

    def sadd(j, sem):
      return pltpu.make_async_copy(ones, deg_sp.at[dst_v.at[j]], sem)

    for j in range(3):
      sadd(j, u[j]).start(add=True)

    def chunk(o, carry):
      for b in range(4):
        j = 4 * o + b
        sadd(j + 3, u[(b + 3) % 4]).start(add=True)
        sadd(j, u[b]).wait()
      return carry
    nr = (kc - 5) // 4
    lax.fori_loop(0, nr, chunk, 0)
    for jj in range(4 * nr, kc):
      if jj + 3 < kc:
        sadd(jj + 3, u[(jj + 3) % 4]).start(add=True)
      sadd(jj, u[jj % 4]).wait()
    plsc.subcore_barrier()

    pltpu.sync_copy(deg_sp.at[pl.ds(start, 640)], zbuf)

    @pl.when(c == 0)
    def _():
      pltpu.sync_copy(zbuf, out0.at[pl.ds(start, 640)])

    @pl.when(c == 1)
    def _():
      pltpu.sync_copy(zbuf, out1.at[pl.ds(start, 640)])

  return deg_kernel


def _make_scatter_kernel(n, d, kc):
  nck = 8
  rck = _CH

  @functools.partial(
      pl.kernel,
      out_type=jax.ShapeDtypeStruct((_NC, n, d), _f32),
      mesh=_sc_mesh(),
      scratch_types=[
          pltpu.VMEM_SHARED((n, d), _f32),
          pltpu.VMEM((kc * _CH,), jnp.int32),
          [pltpu.VMEM((_CH, d), _f32)] * 3,
          [pltpu.VMEM((_CH,), jnp.int32)] * 3,
          [pltpu.SemaphoreType.DMA] * 3,
          [pltpu.SemaphoreType.DMA] * 3,
          [pltpu.SemaphoreType.DMA] * 3,
      ],
  )
  def scatter_kernel(hws, src1, dst1, out, acc_sp, src_v,
                     rows, dc, g, sc, t):
    c = lax.axis_index("c")
    s = lax.axis_index("s")
    wid = s * _NC + c
    ew = kc * _CH
    srcload = lambda: pltpu.make_async_copy(
        src1.at[pl.ds(wid * ew, ew)], src_v, g[0])
    srcload().start()
    _zero_2d(rows[0], rck)
    r0 = jnp.minimum(s * (nck * rck), n - nck * rck)

    def zinit(k):
      return pltpu.make_async_copy(
          rows[0], acc_sp.at[pl.ds(r0 + k * rck, rck)], sc[k % 3])
    for k in range(nck):
      zinit(k).start()
    for k in range(nck):
      zinit(k).wait()
    srcload().wait()

    def gath(j, buf, sem):
      return pltpu.make_async_copy(
          hws.at[src_v.at[pl.ds(j * _CH, _CH)]], buf, sem)

    def stg(j, dcb, sem):
      return pltpu.make_async_copy(
          dst1.at[pl.ds(wid * ew + j * _CH, _CH)], dcb, sem)

    def scat(buf, dcb, sem):
      return pltpu.make_async_copy(buf, acc_sp.at[dcb], sem)

    for b in range(3):
      stg(b, dc[b], t[b]).start()
      gath(b, rows[b], g[b]).start()
    plsc.subcore_barrier()

    def body(o, carry):
      for b in range(3):
        j = 3 * o + b
        gath(j, rows[b], g[b]).wait()
        stg(j, dc[b], t[b]).wait()
        scat(rows[b], dc[b], sc[b]).start(add=True)
        scat(rows[b], dc[b], sc[b]).wait()
        stg(j + 3, dc[b], t[b]).start()
        gath(j + 3, rows[b], g[b]).start()
      return carry
    lax.fori_loop(0, kc // 3 - 1, body, 0)
    for jj in range(3 * (kc // 3 - 1), kc):
      b = jj % 3
      gath(jj, rows[b], g[b]).wait()
      stg(jj, dc[b], t[b]).wait()
      scat(rows[b], dc[b], sc[b]).start(add=True)
      scat(rows[b], dc[b], sc[b]).wait()
      if jj + 3 < kc:
        stg(jj + 3, dc[b], t[b]).start()
        gath(jj + 3, rows[b], g[b]).start()
    plsc.subcore_barrier()

    def cp_in(k, b):
      return pltpu.make_async_copy(
          acc_sp.at[pl.ds(r0 + k * rck, rck)], rows[b], g[b])

    def cp_out(k, b):
      return pltpu.make_async_copy(
          rows[b], out.at[c, pl.ds(r0 + k * rck, rck)], sc[b])

    for k in range(nck):
      b = k % 2
      if k >= 2:
        cp_out(k - 2, b).wait()
      cp_in(k, b).start()
      cp_in(k, b).wait()
      cp_out(k, b).start()
    for k in range(nck - 2, nck):
      cp_out(k, k % 2).wait()

  return scatter_kernel


_PREC = lax.Precision.DEFAULT


def _dinv(c0_ref, c1_ref):
  return lax.rsqrt(c0_ref[...][:, 0] + c1_ref[...][:, 0] + 1.0)


def _mm_scale_body(x_ref, w_ref, c0_ref, c1_ref, hws_ref):
  dinv = _dinv(c0_ref, c1_ref)
  hw = jnp.dot(x_ref[...], w_ref[...],
               preferred_element_type=_f32, precision=_PREC)
  hws_ref[...] = hw * dinv[:, None]


def _mid_body(acc_ref, hws_ref, c0_ref, c1_ref, b_ref, w_ref, out_ref):
  dinv = _dinv(c0_ref, c1_ref)[:, None]
  acc = acc_ref[0] + acc_ref[1]
  h = jnp.maximum(dinv * (acc + hws_ref[...]) + b_ref[...], 0.0)
  hw = jnp.dot(h, w_ref[...], preferred_element_type=_f32, precision=_PREC)
  out_ref[...] = hw * dinv


def _final_body(acc_ref, hws_ref, c0_ref, c1_ref, b_ref, batch_ref, wl_ref,
                bl_ref, out_ref, sums_sc, cnts_sc):
  i = pl.program_id(0)
  dinv = _dinv(c0_ref, c1_ref)[:, None]
  acc = acc_ref[0] + acc_ref[1]
  h = dinv * (acc + hws_ref[...]) + b_ref[...]
  rb = h.shape[0]
  oneh = (batch_ref[...] ==
          lax.broadcasted_iota(jnp.int32, (rb, _G), 1)).astype(_f32)
  dn = (((0,), (0,)), ((), ()))
  part = lax.dot_general(oneh, h, dn, preferred_element_type=_f32,
                         precision=_PREC)
  partc = lax.dot_general(oneh, jnp.ones((rb, h.shape[1]), _f32), dn,
                          preferred_element_type=_f32, precision=_PREC)

  @pl.when(i == 0)
  def _():
    sums_sc[...] = jnp.zeros_like(sums_sc)
    cnts_sc[...] = jnp.zeros_like(cnts_sc)

  sums_sc[...] += part
  cnts_sc[...] += partc

  @pl.when(i == pl.num_programs(0) - 1)
  def _():
    pooled = jnp.maximum(sums_sc[...] / jnp.maximum(cnts_sc[...], 1.0), 0.0)
    out_ref[...] = jnp.dot(pooled, wl_ref[...],
                           preferred_element_type=_f32,
                           precision=_PREC) + bl_ref[...]


def kernel(x, edge_index, batch, W1, b1, W2, b2, Wl, bl):
  n, d = x.shape
  e = edge_index.shape[1]
  h2 = W2.shape[1]
  o = Wl.shape[1]
  ew = e // _NW
  kc = ew // _CH
  assert ew * _NW == e and kc * _CH == ew and n % _NS == 0

  src1 = edge_index[0]
  dst1 = edge_index[1]
  dst3 = dst1.reshape(_NW, kc, _CH)
  batch2d = batch.reshape(n, 1)
  b1r = b1.reshape(1, d)
  b2r = b2.reshape(1, h2)
  blr = bl.reshape(1, o)

  c0, c1 = _make_deg_kernel(n, kc)(dst3)
  c0 = c0.reshape(n, 1)
  c1 = c1.reshape(n, 1)
  scatter = _make_scatter_kernel(n, d, kc)

  rb = 2000
  grid = (n // rb,)
  row_spec = pl.BlockSpec((rb, d), lambda i: (i, 0))
  cnt_spec = pl.BlockSpec((rb, 1), lambda i: (i, 0))
  acc_spec = pl.BlockSpec((_NC, rb, d), lambda i: (0, i, 0))
  w_spec = pl.BlockSpec((d, d), lambda i: (0, 0))
  b_spec = pl.BlockSpec((1, d), lambda i: (0, 0))

  hws1 = pl.pallas_call(
      _mm_scale_body, grid=grid,
      in_specs=[row_spec, w_spec, cnt_spec, cnt_spec],
      out_specs=row_spec,
      out_shape=jax.ShapeDtypeStruct((n, d), _f32),
  )(x, W1, c0, c1)

  acc1 = scatter(hws1, src1, dst1)

  hws2 = pl.pallas_call(
      _mid_body, grid=grid,
      in_specs=[acc_spec, row_spec, cnt_spec, cnt_spec, b_spec, w_spec],
      out_specs=row_spec,
      out_shape=jax.ShapeDtypeStruct((n, h2), _f32),
  )(acc1, hws1, c0, c1, b1r, W2)

  acc2 = scatter(hws2, src1, dst1)

  out = pl.pallas_call(
      _final_body, grid=grid,
      in_specs=[acc_spec, row_spec, cnt_spec, cnt_spec, b_spec,
                pl.BlockSpec((rb, 1), lambda i: (i, 0)),
                pl.BlockSpec((d, o), lambda i: (0, 0)),
                pl.BlockSpec((1, o), lambda i: (0, 0))],
      out_specs=pl.BlockSpec((_G, o), lambda i: (0, 0)),
      out_shape=jax.ShapeDtypeStruct((_G, o), _f32),
      scratch_shapes=[pltpu.VMEM((_G, d), _f32),
                      pltpu.VMEM((_G, d), _f32)],
  )(acc2, hws2, c0, c1, b2r, batch2d, Wl, blr)

  return out


# speedup vs baseline: 1.0044x; 1.0044x over previous
import functools
import jax
import jax.numpy as jnp
from jax import lax
from jax.experimental import pallas as pl
from jax.experimental.pallas import tpu as pltpu
from jax.experimental.pallas import tpu_sc as plsc

_NC = 2
_NS = 16
_NW = _NC * _NS
_CH = 80
_G = 64

_f32 = jnp.float32


def _sc_mesh():
  return plsc.VectorSubcoreMesh(
      core_axis_name="c", subcore_axis_name="s",
      num_cores=_NC, num_subcores=_NS)


def _zero_1d(ref, n):
  def body(i, carry):
    ref[pl.ds(i * 16, 16)] = jnp.zeros((16,), ref.dtype)
    return carry
  lax.fori_loop(0, n // 16, body, 0)


def _fill_1d(ref, n, val):
  def body(i, carry):
    ref[pl.ds(i * 16, 16)] = jnp.full((16,), val, ref.dtype)
    return carry
  lax.fori_loop(0, n // 16, body, 0)
  if n % 16:
    ref[pl.ds(n - 16, 16)] = jnp.full((16,), val, ref.dtype)


def _zero_2d(ref, rows):
  def body(i, carry):
    for k in range(8):
      ref[i, pl.ds(k * 16, 16)] = jnp.zeros((16,), ref.dtype)
    return carry
  lax.fori_loop(0, rows, body, 0)


def _make_deg_kernel(n, kc):
  @functools.partial(
      pl.kernel,
      out_type=(jax.ShapeDtypeStruct((n,), _f32),
                jax.ShapeDtypeStruct((n,), _f32)),
      mesh=_sc_mesh(),
      scratch_types=[
          pltpu.VMEM_SHARED((n,), _f32),
          pltpu.VMEM((kc, _CH), jnp.int32),
          pltpu.VMEM((640,), _f32),
          pltpu.VMEM((_CH,), _f32),
          [pltpu.SemaphoreType.DMA] * 4,
      ],
  )
  def deg_kernel(dst3, out0, out1, deg_sp, dst_v, zbuf, ones, u):
    c = lax.axis_index("c")
    s = lax.axis_index("s")
    wid = s * _NC + c
    _zero_1d(zbuf, 640)
    _fill_1d(ones, _CH, 1.0)
    start = jnp.minimum(s * 640, n - 640)
    pltpu.sync_copy(zbuf, deg_sp.at[pl.ds(start, 640)])
    pltpu.sync_copy(dst3.at[wid], dst_v)
    plsc.subcore_barrier()

    def sadd(j, sem):
      return pltpu.make_async_copy(ones, deg_sp.at[dst_v.at[j]], sem)

    for j in range(3):
      sadd(j, u[j]).start(add=True)

    def chunk(o, carry):
      for b in range(4):
        j = 4 * o + b
        sadd(j + 3, u[(b + 3) % 4]).start(add=True)
        sadd(j, u[b]).wait()
      return carry
    nr = (kc - 5) // 4
    lax.fori_loop(0, nr, chunk, 0)
    for jj in range(4 * nr, kc):
      if jj + 3 < kc:
        sadd(jj + 3, u[(jj + 3) % 4]).start(add=True)
      sadd(jj, u[jj % 4]).wait()
    plsc.subcore_barrier()

    pltpu.sync_copy(deg_sp.at[pl.ds(start, 640)], zbuf)

    @pl.when(c == 0)
    def _():
      pltpu.sync_copy(zbuf, out0.at[pl.ds(start, 640)])

    @pl.when(c == 1)
    def _():
      pltpu.sync_copy(zbuf, out1.at[pl.ds(start, 640)])

  return deg_kernel


def _make_scatter_kernel(n, d, kc):
  nck = 8
  rck = _CH

  @functools.partial(
      pl.kernel,
      out_type=jax.ShapeDtypeStruct((_NC, n, d), _f32),
      mesh=_sc_mesh(),
      scratch_types=[
          pltpu.VMEM_SHARED((n, d), _f32),
          pltpu.VMEM((kc * _CH,), jnp.int32),
          [pltpu.VMEM((_CH, d), _f32)] * 3,
          [pltpu.VMEM((_CH,), jnp.int32)] * 3,
          [pltpu.SemaphoreType.DMA] * 3,
          [pltpu.SemaphoreType.DMA] * 3,
          [pltpu.SemaphoreType.DMA] * 3,
      ],
  )
  def scatter_kernel(hws, src1, dst1, out, acc_sp, src_v,
                     rows, dc, g, sc, t):
    c = lax.axis_index("c")
    s = lax.axis_index("s")
    wid = s * _NC + c
    ew = kc * _CH
    srcload = lambda: pltpu.make_async_copy(
        src1.at[pl.ds(wid * ew, ew)], src_v, g[0])
    srcload().start()
    _zero_2d(rows[0], rck)
    r0 = jnp.minimum(s * (nck * rck), n - nck * rck)

    def zinit(k):
      return pltpu.make_async_copy(
          rows[0], acc_sp.at[pl.ds(r0 + k * rck, rck)], sc[k % 3])
    for k in range(nck):
      zinit(k).start()
    for k in range(nck):
      zinit(k).wait()
    srcload().wait()

    def gath(j, buf, sem):
      return pltpu.make_async_copy(
          hws.at[src_v.at[pl.ds(j * _CH, _CH)]], buf, sem)

    def stg(j, dcb, sem):
      return pltpu.make_async_copy(
          dst1.at[pl.ds(wid * ew + j * _CH, _CH)], dcb, sem)

    def scat(buf, dcb, sem):
      return pltpu.make_async_copy(buf, acc_sp.at[dcb], sem)

    for b in range(3):
      stg(b, dc[b], t[b]).start()
      gath(b, rows[b], g[b]).start()
    plsc.subcore_barrier()

    def body(o, carry):
      for b in range(3):
        j = 3 * o + b
        gath(j, rows[b], g[b]).wait()
        stg(j, dc[b], t[b]).wait()
        scat(rows[b], dc[b], sc[b]).start(add=True)
        scat(rows[b], dc[b], sc[b]).wait()
        stg(j + 3, dc[b], t[b]).start()
        gath(j + 3, rows[b], g[b]).start()
      return carry
    lax.fori_loop(0, kc // 3 - 1, body, 0)
    for jj in range(3 * (kc // 3 - 1), kc):
      b = jj % 3
      gath(jj, rows[b], g[b]).wait()
      stg(jj, dc[b], t[b]).wait()
      scat(rows[b], dc[b], sc[b]).start(add=True)
      scat(rows[b], dc[b], sc[b]).wait()
      if jj + 3 < kc:
        stg(jj + 3, dc[b], t[b]).start()
        gath(jj + 3, rows[b], g[b]).start()
    plsc.subcore_barrier()

    def cp_in(k, b):
      return pltpu.make_async_copy(
          acc_sp.at[pl.ds(r0 + k * rck, rck)], rows[b], g[b])

    def cp_out(k, b):
      return pltpu.make_async_copy(
          rows[b], out.at[c, pl.ds(r0 + k * rck, rck)], sc[b])

    for k in range(nck):
      b = k % 2
      if k >= 2:
        cp_out(k - 2, b).wait()
      cp_in(k, b).start()
      cp_in(k, b).wait()
      cp_out(k, b).start()
    for k in range(nck - 2, nck):
      cp_out(k, k % 2).wait()

  return scatter_kernel


_PREC = lax.Precision.DEFAULT


def _dinv(c0_ref, c1_ref):
  return lax.rsqrt(c0_ref[...][:, 0] + c1_ref[...][:, 0] + 1.0)


def _mm_scale_body(x_ref, w_ref, c0_ref, c1_ref, hws_ref):
  dinv = _dinv(c0_ref, c1_ref)
  hw = jnp.dot(x_ref[...], w_ref[...],
               preferred_element_type=_f32, precision=_PREC)
  hws_ref[...] = hw * dinv[:, None]


def _mid_body(acc_ref, hws_ref, c0_ref, c1_ref, b_ref, w_ref, out_ref):
  dinv = _dinv(c0_ref, c1_ref)[:, None]
  acc = acc_ref[0] + acc_ref[1]
  h = jnp.maximum(dinv * (acc + hws_ref[...]) + b_ref[...], 0.0)
  hw = jnp.dot(h, w_ref[...], preferred_element_type=_f32, precision=_PREC)
  out_ref[...] = hw * dinv


def _final_body(acc_ref, hws_ref, c0_ref, c1_ref, b_ref, batch_ref, wl_ref,
                bl_ref, out_ref, sums_sc, cnts_sc):
  i = pl.program_id(0)
  dinv = _dinv(c0_ref, c1_ref)[:, None]
  acc = acc_ref[0] + acc_ref[1]
  h = dinv * (acc + hws_ref[...]) + b_ref[...]
  rb = h.shape[0]
  oneh = (batch_ref[...] ==
          lax.broadcasted_iota(jnp.int32, (rb, _G), 1)).astype(_f32)
  dn = (((0,), (0,)), ((), ()))
  part = lax.dot_general(oneh, h, dn, preferred_element_type=_f32,
                         precision=_PREC)
  partc = lax.dot_general(oneh, jnp.ones((rb, h.shape[1]), _f32), dn,
                          preferred_element_type=_f32, precision=_PREC)

  @pl.when(i == 0)
  def _():
    sums_sc[...] = jnp.zeros_like(sums_sc)
    cnts_sc[...] = jnp.zeros_like(cnts_sc)

  sums_sc[...] += part
  cnts_sc[...] += partc

  @pl.when(i == pl.num_programs(0) - 1)
  def _():
    pooled = jnp.maximum(sums_sc[...] / jnp.maximum(cnts_sc[...], 1.0), 0.0)
    out_ref[...] = jnp.dot(pooled, wl_ref[...],
                           preferred_element_type=_f32,
                           precision=_PREC) + bl_ref[...]


def kernel(x, edge_index, batch, W1, b1, W2, b2, Wl, bl):
  n, d = x.shape
  e = edge_index.shape[1]
  h2 = W2.shape[1]
  o = Wl.shape[1]
  ew = e // _NW
  kc = ew // _CH
  assert ew * _NW == e and kc * _CH == ew and n % _NS == 0

  src1 = edge_index[0]
  dst1 = edge_index[1]
  dst3 = dst1.reshape(_NW, kc, _CH)
  batch2d = batch.reshape(n, 1)
  b1r = b1.reshape(1, d)
  b2r = b2.reshape(1, h2)
  blr = bl.reshape(1, o)

  c0, c1 = _make_deg_kernel(n, kc)(dst3)
  c0 = c0.reshape(n, 1)
  c1 = c1.reshape(n, 1)
  scatter = _make_scatter_kernel(n, d, kc)

  rb = 5000
  grid = (n // rb,)
  row_spec = pl.BlockSpec((rb, d), lambda i: (i, 0))
  cnt_spec = pl.BlockSpec((rb, 1), lambda i: (i, 0))
  acc_spec = pl.BlockSpec((_NC, rb, d), lambda i: (0, i, 0))
  w_spec = pl.BlockSpec((d, d), lambda i: (0, 0))
  b_spec = pl.BlockSpec((1, d), lambda i: (0, 0))

  hws1 = pl.pallas_call(
      _mm_scale_body, grid=grid,
      in_specs=[row_spec, w_spec, cnt_spec, cnt_spec],
      out_specs=row_spec,
      out_shape=jax.ShapeDtypeStruct((n, d), _f32),
  )(x, W1, c0, c1)

  acc1 = scatter(hws1, src1, dst1)

  hws2 = pl.pallas_call(
      _mid_body, grid=grid,
      in_specs=[acc_spec, row_spec, cnt_spec, cnt_spec, b_spec, w_spec],
      out_specs=row_spec,
      out_shape=jax.ShapeDtypeStruct((n, h2), _f32),
  )(acc1, hws1, c0, c1, b1r, W2)

  acc2 = scatter(hws2, src1, dst1)

  out = pl.pallas_call(
      _final_body, grid=grid,
      in_specs=[acc_spec, row_spec, cnt_spec, cnt_spec, b_spec,
                pl.BlockSpec((rb, 1), lambda i: (i, 0)),
                pl.BlockSpec((d, o), lambda i: (0, 0)),
                pl.BlockSpec((1, o), lambda i: (0, 0))],
      out_specs=pl.BlockSpec((_G, o), lambda i: (0, 0)),
      out_shape=jax.ShapeDtypeStruct((_G, o), _f32),
      scratch_shapes=[pltpu.VMEM((_G, d), _f32),
                      pltpu.VMEM((_G, d), _f32)],
  )(acc2, hws2, c0, c1, b2r, batch2d, Wl, blr)

  return out
